# baseline (speedup 1.0000x reference)
import functools

import jax
import jax.numpy as jnp
from jax import lax
from jax.experimental import pallas as pl
from jax.experimental.pallas import tpu as pltpu
from jax.experimental.pallas import tpu_sc as plsc

N = 10000
D = 128
F = 16
E = 320000
B = 4096
NF = 26
NC, NS = 2, 16
NW = NC * NS
NPAD = 10240
NSLICE = NPAD // NS
NSLICE32 = NPAD // NW
CHUNK = 125
ECHUNKS = E // CHUNK
CPT_HALF = E // NW // CHUNK
CPT_FULL = E // NS // CHUNK
GROUP = 10
XCHUNK = 128
XCH_PER_TILE = B * NF // NW // XCHUNK
XROWS = B * NF // NW

_MESH = plsc.VectorSubcoreMesh(
    core_axis_name="c", subcore_axis_name="s", num_cores=NC, num_subcores=NS)
_SC_PARAMS = pltpu.CompilerParams(
    use_tc_tiling_on_sc=False, needs_layout_passes=False)


def _rsqrt16(x):
    i = plsc.bitcast(x, jnp.int32)
    y = plsc.bitcast(jnp.int32(0x5F3759DF) - (i >> 1), jnp.float32)
    for _ in range(3):
        y = y * (1.5 - 0.5 * x * y * y)
    return y


_MMBLK = 400


def _mm_body(feat_ref, w_ref, h_ref):
    h_ref[...] = jnp.dot(feat_ref[...], w_ref[...],
                         preferred_element_type=jnp.float32)


_mm_call = pl.pallas_call(
    _mm_body,
    grid=(N // _MMBLK,),
    in_specs=[
        pl.BlockSpec((_MMBLK, D), lambda i: (i, 0)),
        pl.BlockSpec((D, F), lambda i: (0, 0)),
    ],
    out_specs=pl.BlockSpec((_MMBLK, F), lambda i: (i, 0)),
    out_shape=jax.ShapeDtypeStruct((NPAD, F), jnp.float32),
)


@functools.partial(
    pl.kernel,
    out_type=jax.ShapeDtypeStruct((NC * NPAD,), jnp.float32),
    mesh=_MESH,
    compiler_params=_SC_PARAMS,
    scratch_types=[
        pltpu.VMEM((CPT_FULL, CHUNK), jnp.int32),
        pltpu.VMEM((128,), jnp.float32),
        pltpu.VMEM((NSLICE,), jnp.float32),
        pltpu.VMEM_SHARED((NPAD,), jnp.float32),
        pltpu.SemaphoreType.DMA,
    ],
)
def _deg_kernel(dst_hbm, deg_hbm, didx, ones, zbuf, deg_sh, sem):
    cid = lax.axis_index("c")
    sid = lax.axis_index("s")
    for i in range(128 // F):
        ones[pl.ds(i * F, F)] = jnp.ones((F,), jnp.float32)
    for i in range(NSLICE // F):
        zbuf[pl.ds(i * F, F)] = jnp.zeros((F,), jnp.float32)
    pltpu.sync_copy(zbuf, deg_sh.at[pl.ds(sid * NSLICE, NSLICE)])
    plsc.subcore_barrier()
    pltpu.sync_copy(dst_hbm.at[pl.ds(sid * CPT_FULL, CPT_FULL)], didx)

    def group_body(gi, carry):
        j0 = gi * GROUP
        descs = []
        for i in range(GROUP):
            descs.append(pltpu.async_copy(
                ones.at[pl.ds(0, CHUNK)], deg_sh.at[didx.at[j0 + i]], sem,
                add=True))
        for d in descs:
            d.wait()
        return carry

    lax.fori_loop(0, CPT_FULL // GROUP, group_body, 0)
    plsc.subcore_barrier()
    pltpu.sync_copy(deg_sh.at[pl.ds(sid * NSLICE, NSLICE)],
                    deg_hbm.at[pl.ds(cid * NPAD + sid * NSLICE, NSLICE)])


@functools.partial(
    pl.kernel,
    out_type=(
        jax.ShapeDtypeStruct((NPAD, F), jnp.float32),
        jax.ShapeDtypeStruct((NPAD,), jnp.float32),
        jax.ShapeDtypeStruct((NC, NPAD, F), jnp.float32),
    ),
    mesh=_MESH,
    compiler_params=_SC_PARAMS,
    scratch_types=[
        pltpu.VMEM((NSLICE,), jnp.float32),
        pltpu.VMEM((NSLICE,), jnp.float32),
        pltpu.VMEM((NSLICE, F), jnp.float32),
        pltpu.VMEM((CPT_HALF, CHUNK), jnp.int32),
        pltpu.VMEM((CPT_HALF, CHUNK), jnp.int32),
        pltpu.VMEM((GROUP, CHUNK, F), jnp.float32),
        pltpu.VMEM_SHARED((NPAD, F), jnp.float32),
        pltpu.SemaphoreType.DMA,
        pltpu.SemaphoreType.DMA,
    ],
)
def _agg_kernel(deg_hbm, h_hbm, src_hbm, dst_hbm, g_hbm, dinv_hbm, acc_hbm,
                degb, dinvb, hb, sidx, didx, rows, acc_sh, gsem, ssem):
    cid = lax.axis_index("c")
    sid = lax.axis_index("s")
    wid = cid * NS + sid
    base = sid * NSLICE
    pltpu.sync_copy(deg_hbm.at[pl.ds(cid * NPAD + base, NSLICE)], degb)

    def rsqrt_body(k, carry):
        v = degb[pl.ds(k * F, F)] + 1.0
        dinvb[pl.ds(k * F, F)] = _rsqrt16(v)
        return carry

    lax.fori_loop(0, NSLICE // F, rsqrt_body, 0)
    pltpu.sync_copy(dinvb, dinv_hbm.at[pl.ds(base, NSLICE)])
    pltpu.sync_copy(h_hbm.at[pl.ds(base, NSLICE)], hb)

    def scale_body(k, carry):
        dv = dinvb[pl.ds(k * F, F)]
        for l in range(F):
            r = k * F + l
            hb[r, :] = hb[r, :] * dv[l]
        return carry

    lax.fori_loop(0, NSLICE // F, scale_body, 0)
    pltpu.sync_copy(hb, g_hbm.at[pl.ds(base, NSLICE)])
    pltpu.sync_copy(hb, acc_sh.at[pl.ds(base, NSLICE)])
    plsc.subcore_barrier()
    pltpu.sync_copy(src_hbm.at[pl.ds(wid * CPT_HALF, CPT_HALF)], sidx)
    pltpu.sync_copy(dst_hbm.at[pl.ds(wid * CPT_HALF, CPT_HALF)], didx)

    def group_body(gi, carry):
        j0 = gi * GROUP
        gd = []
        for i in range(GROUP):
            gd.append(pltpu.async_copy(
                g_hbm.at[sidx.at[j0 + i]], rows.at[i], gsem))
        for d in gd:
            d.wait()
        sd = []
        for i in range(GROUP):
            sd.append(pltpu.async_copy(
                rows.at[i], acc_sh.at[didx.at[j0 + i]], ssem, add=True))
        for d in sd:
            d.wait()
        return carry

    lax.fori_loop(0, CPT_HALF // GROUP, group_body, 0)
    plsc.subcore_barrier()
    pltpu.sync_copy(acc_sh.at[pl.ds(base, NSLICE)],
                    acc_hbm.at[cid, pl.ds(base, NSLICE)])


@functools.partial(
    pl.kernel,
    out_type=jax.ShapeDtypeStruct((NPAD, F), jnp.float32),
    mesh=_MESH,
    compiler_params=_SC_PARAMS,
    scratch_types=[
        pltpu.VMEM((NSLICE32, F), jnp.float32),
        pltpu.VMEM((NSLICE32, F), jnp.float32),
        pltpu.VMEM((NSLICE32, F), jnp.float32),
        pltpu.VMEM((NSLICE32,), jnp.float32),
        pltpu.VMEM((F,), jnp.float32),
    ],
)
def _fin_kernel(acc_hbm, g_hbm, dinv_hbm, b_hbm, out_hbm,
                a0, a1, gb, dinvb, bb):
    cid = lax.axis_index("c")
    sid = lax.axis_index("s")
    wid = cid * NS + sid
    base = wid * NSLICE32
    pltpu.sync_copy(acc_hbm.at[0, pl.ds(base, NSLICE32)], a0)
    pltpu.sync_copy(acc_hbm.at[1, pl.ds(base, NSLICE32)], a1)
    pltpu.sync_copy(g_hbm.at[pl.ds(base, NSLICE32)], gb)
    pltpu.sync_copy(dinv_hbm.at[pl.ds(base, NSLICE32)], dinvb)
    pltpu.sync_copy(b_hbm, bb)
    bv = bb[...]

    def row_body(k, carry):
        dv = dinvb[pl.ds(k * F, F)]
        for l in range(F):
            r = k * F + l
            gb[r, :] = (a0[r, :] + a1[r, :] - gb[r, :]) * dv[l] + bv
        return carry

    lax.fori_loop(0, NSLICE32 // F, row_body, 0)
    pltpu.sync_copy(gb, out_hbm.at[pl.ds(base, NSLICE32)])


@functools.partial(
    pl.kernel,
    out_type=jax.ShapeDtypeStruct((NF, F, B), jnp.float32),
    mesh=_MESH,
    compiler_params=_SC_PARAMS,
    scratch_types=[
        pltpu.VMEM((NF, XCHUNK), jnp.int32),
        pltpu.VMEM((NF, XCHUNK, F), jnp.float32),
        pltpu.VMEM((NF, F, XCHUNK), jnp.float32),
        pltpu.SemaphoreType.DMA,
        pltpu.SemaphoreType.DMA,
    ],
)
def _gather_kernel(out_hbm, xt_hbm, y_hbm, xidx, rows, slabs, gsem, wsem):
    cid = lax.axis_index("c")
    sid = lax.axis_index("s")
    wid = cid * NS + sid
    ibase = wid * XCHUNK
    pltpu.sync_copy(xt_hbm.at[:, pl.ds(ibase, XCHUNK)], xidx)
    lane = lax.iota(jnp.int32, F)

    def transpose_field(jj, carry):
        for e in range(F):
            ev = jnp.full((F,), e, jnp.int32)
            for bb in range(XCHUNK // F):
                v = plsc.load_gather(rows.at[jj], [bb * F + lane, ev])
                slabs[jj, e, pl.ds(bb * F, F)] = v
        return carry

    half = NF // 2
    descs = []
    for j in range(half):
        descs.append(pltpu.async_copy(
            out_hbm.at[xidx.at[j]], rows.at[j], gsem))
    for d in descs:
        d.wait()
    descs = []
    for j in range(half, NF):
        descs.append(pltpu.async_copy(
            out_hbm.at[xidx.at[j]], rows.at[j], gsem))
    lax.fori_loop(0, half, transpose_field, 0)
    for d in descs:
        d.wait()
    lax.fori_loop(half, NF, transpose_field, 0)
    descs = []
    for j in range(NF):
        descs.append(pltpu.async_copy(
            slabs.at[j], y_hbm.at[j, :, pl.ds(ibase, XCHUNK)], wsem))
    for d in descs:
        d.wait()


@jax.jit
def _run(features, train_mat, W, b, x):
    srcr = train_mat[0].reshape(ECHUNKS, CHUNK)
    dstr = train_mat[1].reshape(ECHUNKS, CHUNK)
    h = _mm_call(features, W)
    deg_flat = _deg_kernel(dstr)
    g, dinv, acc_parts = _agg_kernel(deg_flat, h, srcr, dstr)
    out = _fin_kernel(acc_parts, g, dinv, b)
    y = _gather_kernel(out, x.T)
    return jnp.transpose(y, (2, 0, 1))


def kernel(features, train_mat, W, b, x):
    return _run(features, train_mat, W, b, x)


# speedup vs baseline: 1.0563x; 1.0563x over previous
import functools

import jax
import jax.numpy as jnp
from jax import lax
from jax.experimental import pallas as pl
from jax.experimental.pallas import tpu as pltpu
from jax.experimental.pallas import tpu_sc as plsc

N = 10000
D = 128
F = 16
E = 320000
B = 4096
NF = 26
NC, NS = 2, 16
NW = NC * NS
NPAD = 10240
NSLICE = NPAD // NS
NSLICE32 = NPAD // NW
CHUNK = 125
ECHUNKS = E // CHUNK
CPT_HALF = E // NW // CHUNK
CPT_FULL = E // NS // CHUNK
GROUP = 10
XCHUNK = 128
XCH_PER_TILE = B * NF // NW // XCHUNK
XROWS = B * NF // NW

_MESH = plsc.VectorSubcoreMesh(
    core_axis_name="c", subcore_axis_name="s", num_cores=NC, num_subcores=NS)
_SC_PARAMS = pltpu.CompilerParams(
    use_tc_tiling_on_sc=False, needs_layout_passes=False)


def _rsqrt16(x):
    i = plsc.bitcast(x, jnp.int32)
    y = plsc.bitcast(jnp.int32(0x5F3759DF) - (i >> 1), jnp.float32)
    for _ in range(3):
        y = y * (1.5 - 0.5 * x * y * y)
    return y


_MMBLK = 1024


def _mm_body(feat_ref, w_ref, h_ref):
    h_ref[...] = jnp.dot(feat_ref[...], w_ref[...],
                         preferred_element_type=jnp.float32)


_mm_call = pl.pallas_call(
    _mm_body,
    grid=(NPAD // _MMBLK,),
    in_specs=[
        pl.BlockSpec((_MMBLK, D), lambda i: (i, 0)),
        pl.BlockSpec((D, F), lambda i: (0, 0)),
    ],
    out_specs=pl.BlockSpec((_MMBLK, F), lambda i: (i, 0)),
    out_shape=jax.ShapeDtypeStruct((NPAD, F), jnp.float32),
)


@functools.partial(
    pl.kernel,
    out_type=jax.ShapeDtypeStruct((NC * NPAD,), jnp.float32),
    mesh=_MESH,
    compiler_params=_SC_PARAMS,
    scratch_types=[
        pltpu.VMEM((CPT_FULL, CHUNK), jnp.int32),
        pltpu.VMEM((128,), jnp.float32),
        pltpu.VMEM((NSLICE,), jnp.float32),
        pltpu.VMEM_SHARED((NPAD,), jnp.float32),
        pltpu.SemaphoreType.DMA,
    ],
)
def _deg_kernel(dst_hbm, deg_hbm, didx, ones, zbuf, deg_sh, sem):
    cid = lax.axis_index("c")
    sid = lax.axis_index("s")
    for i in range(128 // F):
        ones[pl.ds(i * F, F)] = jnp.ones((F,), jnp.float32)
    for i in range(NSLICE // F):
        zbuf[pl.ds(i * F, F)] = jnp.zeros((F,), jnp.float32)
    pltpu.sync_copy(zbuf, deg_sh.at[pl.ds(sid * NSLICE, NSLICE)])
    plsc.subcore_barrier()
    pltpu.sync_copy(dst_hbm.at[pl.ds(sid * CPT_FULL, CPT_FULL)], didx)

    def group_body(gi, carry):
        j0 = gi * GROUP
        descs = []
        for i in range(GROUP):
            descs.append(pltpu.async_copy(
                ones.at[pl.ds(0, CHUNK)], deg_sh.at[didx.at[j0 + i]], sem,
                add=True))
        for d in descs:
            d.wait()
        return carry

    lax.fori_loop(0, CPT_FULL // GROUP, group_body, 0)
    plsc.subcore_barrier()
    pltpu.sync_copy(deg_sh.at[pl.ds(sid * NSLICE, NSLICE)],
                    deg_hbm.at[pl.ds(cid * NPAD + sid * NSLICE, NSLICE)])


@functools.partial(
    pl.kernel,
    out_type=(
        jax.ShapeDtypeStruct((NPAD, F), jnp.float32),
        jax.ShapeDtypeStruct((NPAD,), jnp.float32),
        jax.ShapeDtypeStruct((NC, NPAD, F), jnp.float32),
    ),
    mesh=_MESH,
    compiler_params=_SC_PARAMS,
    scratch_types=[
        pltpu.VMEM((NSLICE,), jnp.float32),
        pltpu.VMEM((NSLICE,), jnp.float32),
        pltpu.VMEM((NSLICE, F), jnp.float32),
        pltpu.VMEM((CPT_HALF, CHUNK), jnp.int32),
        pltpu.VMEM((CPT_HALF, CHUNK), jnp.int32),
        pltpu.VMEM((GROUP, CHUNK, F), jnp.float32),
        pltpu.VMEM_SHARED((NPAD, F), jnp.float32),
        pltpu.SemaphoreType.DMA,
        pltpu.SemaphoreType.DMA,
    ],
)
def _agg_kernel(deg_hbm, h_hbm, src_hbm, dst_hbm, g_hbm, dinv_hbm, acc_hbm,
                degb, dinvb, hb, sidx, didx, rows, acc_sh, gsem, ssem):
    cid = lax.axis_index("c")
    sid = lax.axis_index("s")
    wid = cid * NS + sid
    base = sid * NSLICE
    pltpu.sync_copy(deg_hbm.at[pl.ds(cid * NPAD + base, NSLICE)], degb)

    def rsqrt_body(k, carry):
        v = degb[pl.ds(k * F, F)] + 1.0
        dinvb[pl.ds(k * F, F)] = _rsqrt16(v)
        return carry

    lax.fori_loop(0, NSLICE // F, rsqrt_body, 0)
    pltpu.sync_copy(dinvb, dinv_hbm.at[pl.ds(base, NSLICE)])
    pltpu.sync_copy(h_hbm.at[pl.ds(base, NSLICE)], hb)

    def scale_body(k, carry):
        dv = dinvb[pl.ds(k * F, F)]
        for l in range(F):
            r = k * F + l
            hb[r, :] = hb[r, :] * dv[l]
        return carry

    lax.fori_loop(0, NSLICE // F, scale_body, 0)
    pltpu.sync_copy(hb, g_hbm.at[pl.ds(base, NSLICE)])
    pltpu.sync_copy(hb, acc_sh.at[pl.ds(base, NSLICE)])
    plsc.subcore_barrier()
    pltpu.sync_copy(src_hbm.at[pl.ds(wid * CPT_HALF, CPT_HALF)], sidx)
    pltpu.sync_copy(dst_hbm.at[pl.ds(wid * CPT_HALF, CPT_HALF)], didx)

    def group_body(gi, carry):
        j0 = gi * GROUP
        gd = []
        for i in range(GROUP):
            gd.append(pltpu.async_copy(
                g_hbm.at[sidx.at[j0 + i]], rows.at[i], gsem))
        for d in gd:
            d.wait()
        sd = []
        for i in range(GROUP):
            sd.append(pltpu.async_copy(
                rows.at[i], acc_sh.at[didx.at[j0 + i]], ssem, add=True))
        for d in sd:
            d.wait()
        return carry

    lax.fori_loop(0, CPT_HALF // GROUP, group_body, 0)
    plsc.subcore_barrier()
    pltpu.sync_copy(acc_sh.at[pl.ds(base, NSLICE)],
                    acc_hbm.at[cid, pl.ds(base, NSLICE)])


@functools.partial(
    pl.kernel,
    out_type=jax.ShapeDtypeStruct((NPAD, F), jnp.float32),
    mesh=_MESH,
    compiler_params=_SC_PARAMS,
    scratch_types=[
        pltpu.VMEM((NSLICE32, F), jnp.float32),
        pltpu.VMEM((NSLICE32, F), jnp.float32),
        pltpu.VMEM((NSLICE32, F), jnp.float32),
        pltpu.VMEM((NSLICE32,), jnp.float32),
        pltpu.VMEM((F,), jnp.float32),
    ],
)
def _fin_kernel(acc_hbm, g_hbm, dinv_hbm, b_hbm, out_hbm,
                a0, a1, gb, dinvb, bb):
    cid = lax.axis_index("c")
    sid = lax.axis_index("s")
    wid = cid * NS + sid
    base = wid * NSLICE32
    pltpu.sync_copy(acc_hbm.at[0, pl.ds(base, NSLICE32)], a0)
    pltpu.sync_copy(acc_hbm.at[1, pl.ds(base, NSLICE32)], a1)
    pltpu.sync_copy(g_hbm.at[pl.ds(base, NSLICE32)], gb)
    pltpu.sync_copy(dinv_hbm.at[pl.ds(base, NSLICE32)], dinvb)
    pltpu.sync_copy(b_hbm, bb)
    bv = bb[...]

    def row_body(k, carry):
        dv = dinvb[pl.ds(k * F, F)]
        for l in range(F):
            r = k * F + l
            gb[r, :] = (a0[r, :] + a1[r, :] - gb[r, :]) * dv[l] + bv
        return carry

    lax.fori_loop(0, NSLICE32 // F, row_body, 0)
    pltpu.sync_copy(gb, out_hbm.at[pl.ds(base, NSLICE32)])


@functools.partial(
    pl.kernel,
    out_type=jax.ShapeDtypeStruct((NF, F, B), jnp.float32),
    mesh=_MESH,
    compiler_params=_SC_PARAMS,
    scratch_types=[
        pltpu.VMEM((B,), jnp.int32),
        pltpu.VMEM((B, F), jnp.float32),
        pltpu.VMEM((F, B // 2), jnp.float32),
        pltpu.SemaphoreType.DMA,
        pltpu.SemaphoreType.DMA,
    ],
)
def _gather_kernel(out_hbm, xt_hbm, y_hbm, xidx, rows, slab, gsem0, gsem1):
    cid = lax.axis_index("c")
    sid = lax.axis_index("s")
    wid = cid * NS + sid
    lane = lax.iota(jnp.int32, F)
    half = B // 2

    @pl.when(wid < NF)
    def _():
        pltpu.sync_copy(xt_hbm.at[wid], xidx)
        for k in range(16):
            pltpu.async_copy(
                out_hbm.at[xidx.at[pl.ds(k * XCHUNK, XCHUNK)]],
                rows.at[pl.ds(k * XCHUNK, XCHUNK)], gsem0)
        descs1 = []
        for k in range(16, 32):
            descs1.append(pltpu.async_copy(
                out_hbm.at[xidx.at[pl.ds(k * XCHUNK, XCHUNK)]],
                rows.at[pl.ds(k * XCHUNK, XCHUNK)], gsem1))
        pltpu.make_async_copy(
            out_hbm.at[pl.ds(0, half)], rows.at[pl.ds(0, half)], gsem0).wait()

        def make_tbody(hh):
            def tbody(q, carry):
                ridx = hh * half + q * F + lane
                for e in range(F):
                    v = plsc.load_gather(
                        rows, [ridx, jnp.full((F,), e, jnp.int32)])
                    slab[e, pl.ds(q * F, F)] = v
                return carry
            return tbody

        lax.fori_loop(0, half // F, make_tbody(0), 0)
        pltpu.sync_copy(slab, y_hbm.at[wid, :, pl.ds(0, half)])
        for d in descs1:
            d.wait()
        lax.fori_loop(0, half // F, make_tbody(1), 0)
        pltpu.sync_copy(slab, y_hbm.at[wid, :, pl.ds(half, half)])


@jax.jit
def _run(features, train_mat, W, b, x):
    srcr = train_mat[0].reshape(ECHUNKS, CHUNK)
    dstr = train_mat[1].reshape(ECHUNKS, CHUNK)
    h = _mm_call(features, W)
    deg_flat = _deg_kernel(dstr)
    g, dinv, acc_parts = _agg_kernel(deg_flat, h, srcr, dstr)
    out = _fin_kernel(acc_parts, g, dinv, b)
    y = _gather_kernel(out, x.T)
    return jnp.transpose(y, (2, 0, 1))


def kernel(features, train_mat, W, b, x):
    return _run(features, train_mat, W, b, x)


# speedup vs baseline: 1.1636x; 1.1016x over previous
import functools

import jax
import jax.numpy as jnp
from jax import lax
from jax.experimental import pallas as pl
from jax.experimental.pallas import tpu as pltpu
from jax.experimental.pallas import tpu_sc as plsc

N = 10000
D = 128
F = 16
E = 320000
B = 4096
NF = 26
NC, NS = 2, 16
NW = NC * NS
NPAD = 10240
NSLICE = NPAD // NS
NSLICE32 = NPAD // NW
CHUNK = 125
ECHUNKS = E // CHUNK
CPT_HALF = E // NW // CHUNK
CPT_FULL = E // NS // CHUNK
GROUP = 10
XCHUNK = 128
XCH_PER_TILE = B * NF // NW // XCHUNK
XROWS = B * NF // NW

_MESH = plsc.VectorSubcoreMesh(
    core_axis_name="c", subcore_axis_name="s", num_cores=NC, num_subcores=NS)
_SC_PARAMS = pltpu.CompilerParams(
    use_tc_tiling_on_sc=False, needs_layout_passes=False)


def _rsqrt16(x):
    i = plsc.bitcast(x, jnp.int32)
    y = plsc.bitcast(jnp.int32(0x5F3759DF) - (i >> 1), jnp.float32)
    for _ in range(3):
        y = y * (1.5 - 0.5 * x * y * y)
    return y


_MMBLK = 1024


def _mm_body(feat_ref, w_ref, h_ref):
    h_ref[...] = jnp.dot(feat_ref[...], w_ref[...],
                         preferred_element_type=jnp.float32)


_mm_call = pl.pallas_call(
    _mm_body,
    grid=(NPAD // _MMBLK,),
    in_specs=[
        pl.BlockSpec((_MMBLK, D), lambda i: (i, 0)),
        pl.BlockSpec((D, F), lambda i: (0, 0)),
    ],
    out_specs=pl.BlockSpec((_MMBLK, F), lambda i: (i, 0)),
    out_shape=jax.ShapeDtypeStruct((NPAD, F), jnp.float32),
)


@functools.partial(
    pl.kernel,
    out_type=jax.ShapeDtypeStruct((NC * NPAD,), jnp.float32),
    mesh=_MESH,
    compiler_params=_SC_PARAMS,
    scratch_types=[
        pltpu.VMEM((CPT_FULL, CHUNK), jnp.int32),
        pltpu.VMEM((128,), jnp.float32),
        pltpu.VMEM((NSLICE,), jnp.float32),
        pltpu.VMEM_SHARED((NPAD,), jnp.float32),
        pltpu.SemaphoreType.DMA,
    ],
)
def _deg_kernel(dst_hbm, deg_hbm, didx, ones, zbuf, deg_sh, sem):
    cid = lax.axis_index("c")
    sid = lax.axis_index("s")
    for i in range(128 // F):
        ones[pl.ds(i * F, F)] = jnp.ones((F,), jnp.float32)
    for i in range(NSLICE // F):
        zbuf[pl.ds(i * F, F)] = jnp.zeros((F,), jnp.float32)
    pltpu.sync_copy(zbuf, deg_sh.at[pl.ds(sid * NSLICE, NSLICE)])
    plsc.subcore_barrier()
    pltpu.sync_copy(dst_hbm.at[pl.ds(sid * CPT_FULL, CPT_FULL)], didx)

    def group_body(gi, carry):
        j0 = gi * GROUP
        descs = []
        for i in range(GROUP):
            descs.append(pltpu.async_copy(
                ones.at[pl.ds(0, CHUNK)], deg_sh.at[didx.at[j0 + i]], sem,
                add=True))
        for d in descs:
            d.wait()
        return carry

    lax.fori_loop(0, CPT_FULL // GROUP, group_body, 0)
    plsc.subcore_barrier()
    pltpu.sync_copy(deg_sh.at[pl.ds(sid * NSLICE, NSLICE)],
                    deg_hbm.at[pl.ds(cid * NPAD + sid * NSLICE, NSLICE)])


@functools.partial(
    pl.kernel,
    out_type=(
        jax.ShapeDtypeStruct((NPAD, F), jnp.float32),
        jax.ShapeDtypeStruct((NPAD,), jnp.float32),
        jax.ShapeDtypeStruct((NC, NPAD, F), jnp.float32),
    ),
    mesh=_MESH,
    compiler_params=_SC_PARAMS,
    scratch_types=[
        pltpu.VMEM((NSLICE,), jnp.float32),
        pltpu.VMEM((NSLICE,), jnp.float32),
        pltpu.VMEM((NSLICE, F), jnp.float32),
        pltpu.VMEM((CPT_HALF, CHUNK), jnp.int32),
        pltpu.VMEM((CPT_HALF, CHUNK), jnp.int32),
        pltpu.VMEM((GROUP, CHUNK, F), jnp.float32),
        pltpu.VMEM_SHARED((NPAD, F), jnp.float32),
        pltpu.SemaphoreType.DMA,
        pltpu.SemaphoreType.DMA,
    ],
)
def _agg_kernel(deg_hbm, h_hbm, src_hbm, dst_hbm, g_hbm, dinv_hbm, acc_hbm,
                degb, dinvb, hb, sidx, didx, rows, acc_sh, gsem, ssem):
    cid = lax.axis_index("c")
    sid = lax.axis_index("s")
    wid = cid * NS + sid
    base = sid * NSLICE
    pltpu.sync_copy(deg_hbm.at[pl.ds(cid * NPAD + base, NSLICE)], degb)

    def rsqrt_body(k, carry):
        v = degb[pl.ds(k * F, F)] + 1.0
        dinvb[pl.ds(k * F, F)] = _rsqrt16(v)
        return carry

    lax.fori_loop(0, NSLICE // F, rsqrt_body, 0)
    pltpu.sync_copy(dinvb, dinv_hbm.at[pl.ds(base, NSLICE)])
    pltpu.sync_copy(h_hbm.at[pl.ds(base, NSLICE)], hb)

    def scale_body(k, carry):
        dv = dinvb[pl.ds(k * F, F)]
        for l in range(F):
            r = k * F + l
            hb[r, :] = hb[r, :] * dv[l]
        return carry

    lax.fori_loop(0, NSLICE // F, scale_body, 0)
    pltpu.sync_copy(hb, g_hbm.at[pl.ds(base, NSLICE)])
    pltpu.sync_copy(hb, acc_sh.at[pl.ds(base, NSLICE)])
    plsc.subcore_barrier()
    pltpu.sync_copy(src_hbm.at[pl.ds(wid * CPT_HALF, CPT_HALF)], sidx)
    pltpu.sync_copy(dst_hbm.at[pl.ds(wid * CPT_HALF, CPT_HALF)], didx)

    def group_body(gi, carry):
        j0 = gi * GROUP
        gd = []
        for i in range(GROUP):
            gd.append(pltpu.async_copy(
                g_hbm.at[sidx.at[j0 + i]], rows.at[i], gsem))
        for d in gd:
            d.wait()
        sd = []
        for i in range(GROUP):
            sd.append(pltpu.async_copy(
                rows.at[i], acc_sh.at[didx.at[j0 + i]], ssem, add=True))
        for d in sd:
            d.wait()
        return carry

    lax.fori_loop(0, CPT_HALF // GROUP, group_body, 0)
    plsc.subcore_barrier()
    pltpu.sync_copy(acc_sh.at[pl.ds(base, NSLICE)],
                    acc_hbm.at[cid, pl.ds(base, NSLICE)])


@functools.partial(
    pl.kernel,
    out_type=jax.ShapeDtypeStruct((NPAD, F), jnp.float32),
    mesh=_MESH,
    compiler_params=_SC_PARAMS,
    scratch_types=[
        pltpu.VMEM((NSLICE32, F), jnp.float32),
        pltpu.VMEM((NSLICE32, F), jnp.float32),
        pltpu.VMEM((NSLICE32, F), jnp.float32),
        pltpu.VMEM((NSLICE32,), jnp.float32),
        pltpu.VMEM((F,), jnp.float32),
    ],
)
def _fin_kernel(acc_hbm, g_hbm, dinv_hbm, b_hbm, out_hbm,
                a0, a1, gb, dinvb, bb):
    cid = lax.axis_index("c")
    sid = lax.axis_index("s")
    wid = cid * NS + sid
    base = wid * NSLICE32
    pltpu.sync_copy(acc_hbm.at[0, pl.ds(base, NSLICE32)], a0)
    pltpu.sync_copy(acc_hbm.at[1, pl.ds(base, NSLICE32)], a1)
    pltpu.sync_copy(g_hbm.at[pl.ds(base, NSLICE32)], gb)
    pltpu.sync_copy(dinv_hbm.at[pl.ds(base, NSLICE32)], dinvb)
    pltpu.sync_copy(b_hbm, bb)
    bv = bb[...]

    def row_body(k, carry):
        dv = dinvb[pl.ds(k * F, F)]
        for l in range(F):
            r = k * F + l
            gb[r, :] = (a0[r, :] + a1[r, :] - gb[r, :]) * dv[l] + bv
        return carry

    lax.fori_loop(0, NSLICE32 // F, row_body, 0)
    pltpu.sync_copy(gb, out_hbm.at[pl.ds(base, NSLICE32)])


@functools.partial(
    pl.kernel,
    out_type=jax.ShapeDtypeStruct((NF, F, B), jnp.float32),
    mesh=_MESH,
    compiler_params=_SC_PARAMS,
    scratch_types=[
        pltpu.VMEM((B,), jnp.int32),
        pltpu.VMEM((B, F), jnp.float32),
        pltpu.VMEM((F, B // 2), jnp.float32),
        pltpu.SemaphoreType.DMA,
        pltpu.SemaphoreType.DMA,
    ],
)
def _gather_kernel(out_hbm, xt_hbm, y_hbm, xidx, rows, slab, gsem0, gsem1):
    cid = lax.axis_index("c")
    sid = lax.axis_index("s")
    wid = cid * NS + sid
    lane = lax.iota(jnp.int32, F)
    half = B // 2

    @pl.when(wid < NF)
    def _():
        pltpu.sync_copy(xt_hbm.at[wid], xidx)
        for k in range(16):
            pltpu.async_copy(
                out_hbm.at[xidx.at[pl.ds(k * XCHUNK, XCHUNK)]],
                rows.at[pl.ds(k * XCHUNK, XCHUNK)], gsem0)
        descs1 = []
        for k in range(16, 32):
            descs1.append(pltpu.async_copy(
                out_hbm.at[xidx.at[pl.ds(k * XCHUNK, XCHUNK)]],
                rows.at[pl.ds(k * XCHUNK, XCHUNK)], gsem1))
        pltpu.make_async_copy(
            out_hbm.at[pl.ds(0, half)], rows.at[pl.ds(0, half)], gsem0).wait()

        diags = [(lane + d) & (F - 1) for d in range(F)]

        def make_tbody(hh):
            def tbody(q, carry):
                ridx = hh * half + q * F + lane
                cidx = q * F + lane
                for d in range(F):
                    v = plsc.load_gather(rows, [ridx, diags[d]])
                    plsc.store_scatter(slab, [diags[d], cidx], v)
                return carry
            return tbody

        lax.fori_loop(0, half // F, make_tbody(0), 0)
        pltpu.sync_copy(slab, y_hbm.at[wid, :, pl.ds(0, half)])
        for d in descs1:
            d.wait()
        lax.fori_loop(0, half // F, make_tbody(1), 0)
        pltpu.sync_copy(slab, y_hbm.at[wid, :, pl.ds(half, half)])


@jax.jit
def _run(features, train_mat, W, b, x):
    srcr = train_mat[0].reshape(ECHUNKS, CHUNK)
    dstr = train_mat[1].reshape(ECHUNKS, CHUNK)
    h = _mm_call(features, W)
    deg_flat = _deg_kernel(dstr)
    g, dinv, acc_parts = _agg_kernel(deg_flat, h, srcr, dstr)
    out = _fin_kernel(acc_parts, g, dinv, b)
    y = _gather_kernel(out, x.T)
    return jnp.transpose(y, (2, 0, 1))


def kernel(features, train_mat, W, b, x):
    return _run(features, train_mat, W, b, x)


# speedup vs baseline: 1.2951x; 1.1130x over previous
import functools

import jax
import jax.numpy as jnp
from jax import lax
from jax.experimental import pallas as pl
from jax.experimental.pallas import tpu as pltpu
from jax.experimental.pallas import tpu_sc as plsc

N = 10000
D = 128
F = 16
E = 320000
B = 4096
NF = 26
NC, NS = 2, 16
NW = NC * NS
NPAD = 10240
NSLICE = NPAD // NS
NSLICE32 = NPAD // NW
CHUNK = 125
ECHUNKS = E // CHUNK
CPT_HALF = E // NW // CHUNK
CPT_FULL = E // NS // CHUNK
GROUP = 16
XCHUNK = 128
XCH_PER_TILE = B * NF // NW // XCHUNK
XROWS = B * NF // NW

_MESH = plsc.VectorSubcoreMesh(
    core_axis_name="c", subcore_axis_name="s", num_cores=NC, num_subcores=NS)
_SC_PARAMS = pltpu.CompilerParams(
    use_tc_tiling_on_sc=False, needs_layout_passes=False)


def _rsqrt16(x):
    i = plsc.bitcast(x, jnp.int32)
    y = plsc.bitcast(jnp.int32(0x5F3759DF) - (i >> 1), jnp.float32)
    for _ in range(3):
        y = y * (1.5 - 0.5 * x * y * y)
    return y


_MMBLK = 1024


def _mm_body(feat_ref, w_ref, h_ref):
    h_ref[...] = jnp.dot(feat_ref[...], w_ref[...],
                         preferred_element_type=jnp.float32)


_mm_call = pl.pallas_call(
    _mm_body,
    grid=(NPAD // _MMBLK,),
    in_specs=[
        pl.BlockSpec((_MMBLK, D), lambda i: (i, 0)),
        pl.BlockSpec((D, F), lambda i: (0, 0)),
    ],
    out_specs=pl.BlockSpec((_MMBLK, F), lambda i: (i, 0)),
    out_shape=jax.ShapeDtypeStruct((NPAD, F), jnp.float32),
)


@functools.partial(
    pl.kernel,
    out_type=jax.ShapeDtypeStruct((NC * NPAD,), jnp.float32),
    mesh=_MESH,
    compiler_params=_SC_PARAMS,
    scratch_types=[
        pltpu.VMEM((CPT_FULL, CHUNK), jnp.int32),
        pltpu.VMEM((128,), jnp.float32),
        pltpu.VMEM((NSLICE,), jnp.float32),
        pltpu.VMEM_SHARED((NPAD,), jnp.float32),
        pltpu.SemaphoreType.DMA,
    ],
)
def _deg_kernel(tm_hbm, deg_hbm, didx, ones, zbuf, deg_sh, sem):
    cid = lax.axis_index("c")
    sid = lax.axis_index("s")
    for i in range(128 // F):
        ones[pl.ds(i * F, F)] = jnp.ones((F,), jnp.float32)
    for i in range(NSLICE // F):
        zbuf[pl.ds(i * F, F)] = jnp.zeros((F,), jnp.float32)
    pltpu.sync_copy(zbuf, deg_sh.at[pl.ds(sid * NSLICE, NSLICE)])
    plsc.subcore_barrier()
    pltpu.sync_copy(tm_hbm.at[1, pl.ds(sid * CPT_FULL, CPT_FULL)], didx)

    def group_body(gi, carry):
        j0 = gi * GROUP
        descs = []
        for i in range(GROUP):
            descs.append(pltpu.async_copy(
                ones.at[pl.ds(0, CHUNK)], deg_sh.at[didx.at[j0 + i]], sem,
                add=True))
        for d in descs:
            d.wait()
        return carry

    lax.fori_loop(0, CPT_FULL // GROUP, group_body, 0)
    plsc.subcore_barrier()
    pltpu.sync_copy(deg_sh.at[pl.ds(sid * NSLICE, NSLICE)],
                    deg_hbm.at[pl.ds(cid * NPAD + sid * NSLICE, NSLICE)])


@functools.partial(
    pl.kernel,
    out_type=(
        jax.ShapeDtypeStruct((NPAD, F), jnp.float32),
        jax.ShapeDtypeStruct((NPAD,), jnp.float32),
        jax.ShapeDtypeStruct((NC, NPAD, F), jnp.float32),
    ),
    mesh=_MESH,
    compiler_params=_SC_PARAMS,
    scratch_types=[
        pltpu.VMEM((NSLICE,), jnp.float32),
        pltpu.VMEM((NSLICE,), jnp.float32),
        pltpu.VMEM((NSLICE, F), jnp.float32),
        pltpu.VMEM((CPT_HALF, CHUNK), jnp.int32),
        pltpu.VMEM((CPT_HALF, CHUNK), jnp.int32),
        pltpu.VMEM((GROUP, CHUNK, F), jnp.float32),
        pltpu.VMEM_SHARED((NPAD, F), jnp.float32),
        pltpu.SemaphoreType.DMA,
        pltpu.SemaphoreType.DMA,
    ],
)
def _agg_kernel(deg_hbm, h_hbm, tm_hbm, g_hbm, dinv_hbm, acc_hbm,
                degb, dinvb, hb, sidx, didx, rows, acc_sh, gsem, ssem):
    cid = lax.axis_index("c")
    sid = lax.axis_index("s")
    wid = cid * NS + sid
    base = sid * NSLICE
    pltpu.sync_copy(deg_hbm.at[pl.ds(cid * NPAD + base, NSLICE)], degb)

    def rsqrt_body(k, carry):
        v = degb[pl.ds(k * F, F)] + 1.0
        dinvb[pl.ds(k * F, F)] = _rsqrt16(v)
        return carry

    lax.fori_loop(0, NSLICE // F, rsqrt_body, 0)
    pltpu.sync_copy(dinvb, dinv_hbm.at[pl.ds(base, NSLICE)])
    pltpu.sync_copy(h_hbm.at[pl.ds(base, NSLICE)], hb)

    def scale_body(k, carry):
        dv = dinvb[pl.ds(k * F, F)]
        for l in range(F):
            r = k * F + l
            hb[r, :] = hb[r, :] * dv[l]
        return carry

    lax.fori_loop(0, NSLICE // F, scale_body, 0)
    pltpu.sync_copy(hb, g_hbm.at[pl.ds(base, NSLICE)])
    pltpu.sync_copy(hb, acc_sh.at[pl.ds(base, NSLICE)])
    plsc.subcore_barrier()
    pltpu.sync_copy(tm_hbm.at[0, pl.ds(wid * CPT_HALF, CPT_HALF)], sidx)
    pltpu.sync_copy(tm_hbm.at[1, pl.ds(wid * CPT_HALF, CPT_HALF)], didx)

    def group_body(gi, carry):
        j0 = gi * GROUP
        gd = []
        for i in range(GROUP):
            gd.append(pltpu.async_copy(
                g_hbm.at[sidx.at[j0 + i]], rows.at[i], gsem))
        for d in gd:
            d.wait()
        sd = []
        for i in range(GROUP):
            sd.append(pltpu.async_copy(
                rows.at[i], acc_sh.at[didx.at[j0 + i]], ssem, add=True))
        for d in sd:
            d.wait()
        return carry

    lax.fori_loop(0, CPT_HALF // GROUP, group_body, 0)
    plsc.subcore_barrier()
    pltpu.sync_copy(acc_sh.at[pl.ds(base, NSLICE)],
                    acc_hbm.at[cid, pl.ds(base, NSLICE)])


@functools.partial(
    pl.kernel,
    out_type=jax.ShapeDtypeStruct((NPAD, F), jnp.float32),
    mesh=_MESH,
    compiler_params=_SC_PARAMS,
    scratch_types=[
        pltpu.VMEM((NSLICE32, F), jnp.float32),
        pltpu.VMEM((NSLICE32, F), jnp.float32),
        pltpu.VMEM((NSLICE32, F), jnp.float32),
        pltpu.VMEM((NSLICE32,), jnp.float32),
        pltpu.VMEM((F,), jnp.float32),
    ],
)
def _fin_kernel(acc_hbm, g_hbm, dinv_hbm, b_hbm, out_hbm,
                a0, a1, gb, dinvb, bb):
    cid = lax.axis_index("c")
    sid = lax.axis_index("s")
    wid = cid * NS + sid
    base = wid * NSLICE32
    pltpu.sync_copy(acc_hbm.at[0, pl.ds(base, NSLICE32)], a0)
    pltpu.sync_copy(acc_hbm.at[1, pl.ds(base, NSLICE32)], a1)
    pltpu.sync_copy(g_hbm.at[pl.ds(base, NSLICE32)], gb)
    pltpu.sync_copy(dinv_hbm.at[pl.ds(base, NSLICE32)], dinvb)
    pltpu.sync_copy(b_hbm, bb)
    bv = bb[...]

    def row_body(k, carry):
        dv = dinvb[pl.ds(k * F, F)]
        for l in range(F):
            r = k * F + l
            gb[r, :] = (a0[r, :] + a1[r, :] - gb[r, :]) * dv[l] + bv
        return carry

    lax.fori_loop(0, NSLICE32 // F, row_body, 0)
    pltpu.sync_copy(gb, out_hbm.at[pl.ds(base, NSLICE32)])


@functools.partial(
    pl.kernel,
    out_type=jax.ShapeDtypeStruct((NF, F, B), jnp.float32),
    mesh=_MESH,
    compiler_params=_SC_PARAMS,
    scratch_types=[
        pltpu.VMEM((B,), jnp.int32),
        pltpu.VMEM((B, F), jnp.float32),
        pltpu.VMEM((F, B // 2), jnp.float32),
        pltpu.SemaphoreType.DMA,
        pltpu.SemaphoreType.DMA,
    ],
)
def _gather_kernel(out_hbm, xt_hbm, y_hbm, xidx, rows, slab, gsem0, gsem1):
    cid = lax.axis_index("c")
    sid = lax.axis_index("s")
    wid = cid * NS + sid
    lane = lax.iota(jnp.int32, F)
    half = B // 2

    @pl.when(wid < NF)
    def _():
        pltpu.sync_copy(xt_hbm.at[wid], xidx)
        for k in range(16):
            pltpu.async_copy(
                out_hbm.at[xidx.at[pl.ds(k * XCHUNK, XCHUNK)]],
                rows.at[pl.ds(k * XCHUNK, XCHUNK)], gsem0)
        descs1 = []
        for k in range(16, 32):
            descs1.append(pltpu.async_copy(
                out_hbm.at[xidx.at[pl.ds(k * XCHUNK, XCHUNK)]],
                rows.at[pl.ds(k * XCHUNK, XCHUNK)], gsem1))
        pltpu.make_async_copy(
            out_hbm.at[pl.ds(0, half)], rows.at[pl.ds(0, half)], gsem0).wait()

        diags = [(lane + d) & (F - 1) for d in range(F)]

        def make_tbody(hh):
            def tbody(q, carry):
                ridx = hh * half + q * F + lane
                cidx = q * F + lane
                for d in range(F):
                    v = plsc.load_gather(rows, [ridx, diags[d]])
                    plsc.store_scatter(slab, [diags[d], cidx], v)
                return carry
            return tbody

        lax.fori_loop(0, half // F, make_tbody(0), 0)
        pltpu.sync_copy(slab, y_hbm.at[wid, :, pl.ds(0, half)])
        for d in descs1:
            d.wait()
        lax.fori_loop(0, half // F, make_tbody(1), 0)
        pltpu.sync_copy(slab, y_hbm.at[wid, :, pl.ds(half, half)])


@jax.jit
def _run(features, train_mat, W, b, x):
    tmr = train_mat.reshape(2, ECHUNKS, CHUNK)
    h = _mm_call(features, W)
    deg_flat = _deg_kernel(tmr)
    g, dinv, acc_parts = _agg_kernel(deg_flat, h, tmr)
    out = _fin_kernel(acc_parts, g, dinv, b)
    y = _gather_kernel(out, x.T)
    return jnp.transpose(y, (2, 0, 1))


def kernel(features, train_mat, W, b, x):
    return _run(features, train_mat, W, b, x)


# speedup vs baseline: 1.3592x; 1.0495x over previous
import functools

import jax
import jax.numpy as jnp
from jax import lax
from jax.experimental import pallas as pl
from jax.experimental.pallas import tpu as pltpu
from jax.experimental.pallas import tpu_sc as plsc

N = 10000
D = 128
F = 16
E = 320000
B = 4096
NF = 26
NC, NS = 2, 16
NW = NC * NS
NPAD = 10240
NSLICE = NPAD // NS
NSLICE32 = NPAD // NW
CHUNK = 125
ECHUNKS = E // CHUNK
CPT_HALF = E // NW // CHUNK
CPT_FULL = E // NS // CHUNK
GROUP = 16
XCHUNK = 128
XCH_PER_TILE = B * NF // NW // XCHUNK
XROWS = B * NF // NW

_MESH = plsc.VectorSubcoreMesh(
    core_axis_name="c", subcore_axis_name="s", num_cores=NC, num_subcores=NS)
_SC_PARAMS = pltpu.CompilerParams(
    use_tc_tiling_on_sc=False, needs_layout_passes=False)


def _rsqrt16(x):
    i = plsc.bitcast(x, jnp.int32)
    y = plsc.bitcast(jnp.int32(0x5F3759DF) - (i >> 1), jnp.float32)
    for _ in range(3):
        y = y * (1.5 - 0.5 * x * y * y)
    return y


_MMBLK = 1024


def _mm_body(feat_ref, w_ref, h_ref):
    h_ref[...] = jnp.dot(feat_ref[...], w_ref[...],
                         preferred_element_type=jnp.float32)


_mm_call = pl.pallas_call(
    _mm_body,
    grid=(NPAD // _MMBLK,),
    in_specs=[
        pl.BlockSpec((_MMBLK, D), lambda i: (i, 0)),
        pl.BlockSpec((D, F), lambda i: (0, 0)),
    ],
    out_specs=pl.BlockSpec((_MMBLK, F), lambda i: (i, 0)),
    out_shape=jax.ShapeDtypeStruct((NPAD, F), jnp.float32),
)


@functools.partial(
    pl.kernel,
    out_type=jax.ShapeDtypeStruct((NC * NPAD,), jnp.float32),
    mesh=_MESH,
    compiler_params=_SC_PARAMS,
    scratch_types=[
        pltpu.VMEM((CPT_FULL, CHUNK), jnp.int32),
        pltpu.VMEM((128,), jnp.float32),
        pltpu.VMEM((NSLICE,), jnp.float32),
        pltpu.VMEM_SHARED((NPAD,), jnp.float32),
        pltpu.SemaphoreType.DMA,
    ],
)
def _deg_kernel(tm_hbm, deg_hbm, didx, ones, zbuf, deg_sh, sem):
    cid = lax.axis_index("c")
    sid = lax.axis_index("s")
    for i in range(128 // F):
        ones[pl.ds(i * F, F)] = jnp.ones((F,), jnp.float32)
    for i in range(NSLICE // F):
        zbuf[pl.ds(i * F, F)] = jnp.zeros((F,), jnp.float32)
    pltpu.sync_copy(zbuf, deg_sh.at[pl.ds(sid * NSLICE, NSLICE)])
    plsc.subcore_barrier()
    pltpu.sync_copy(tm_hbm.at[1, pl.ds(sid * CPT_FULL, CPT_FULL)], didx)

    def group_body(gi, carry):
        j0 = gi * GROUP
        descs = []
        for i in range(GROUP):
            descs.append(pltpu.async_copy(
                ones.at[pl.ds(0, CHUNK)], deg_sh.at[didx.at[j0 + i]], sem,
                add=True))
        for d in descs:
            d.wait()
        return carry

    lax.fori_loop(0, CPT_FULL // GROUP, group_body, 0)
    plsc.subcore_barrier()
    pltpu.sync_copy(deg_sh.at[pl.ds(sid * NSLICE, NSLICE)],
                    deg_hbm.at[pl.ds(cid * NPAD + sid * NSLICE, NSLICE)])


@functools.partial(
    pl.kernel,
    out_type=(
        jax.ShapeDtypeStruct((NPAD, F), jnp.float32),
        jax.ShapeDtypeStruct((NPAD,), jnp.float32),
        jax.ShapeDtypeStruct((NC, NPAD, F), jnp.float32),
    ),
    mesh=_MESH,
    compiler_params=_SC_PARAMS,
    scratch_types=[
        pltpu.VMEM((NSLICE,), jnp.float32),
        pltpu.VMEM((NSLICE,), jnp.float32),
        pltpu.VMEM((NSLICE, F), jnp.float32),
        pltpu.VMEM((CPT_HALF, CHUNK), jnp.int32),
        pltpu.VMEM((CPT_HALF, CHUNK), jnp.int32),
        pltpu.VMEM((2, GROUP, CHUNK, F), jnp.float32),
        pltpu.VMEM_SHARED((NPAD, F), jnp.float32),
        pltpu.SemaphoreType.DMA,
        pltpu.SemaphoreType.DMA,
        pltpu.SemaphoreType.DMA,
    ],
)
def _agg_kernel(deg_hbm, h_hbm, tm_hbm, g_hbm, dinv_hbm, acc_hbm,
                degb, dinvb, hb, sidx, didx, rows, acc_sh, gsem, ssem0, ssem1):
    cid = lax.axis_index("c")
    sid = lax.axis_index("s")
    wid = cid * NS + sid
    base = sid * NSLICE
    pltpu.sync_copy(deg_hbm.at[pl.ds(cid * NPAD + base, NSLICE)], degb)

    def rsqrt_body(k, carry):
        v = degb[pl.ds(k * F, F)] + 1.0
        dinvb[pl.ds(k * F, F)] = _rsqrt16(v)
        return carry

    lax.fori_loop(0, NSLICE // F, rsqrt_body, 0)
    pltpu.sync_copy(dinvb, dinv_hbm.at[pl.ds(base, NSLICE)])
    pltpu.sync_copy(h_hbm.at[pl.ds(base, NSLICE)], hb)

    def scale_body(k, carry):
        dv = dinvb[pl.ds(k * F, F)]
        for l in range(F):
            r = k * F + l
            hb[r, :] = hb[r, :] * dv[l]
        return carry

    lax.fori_loop(0, NSLICE // F, scale_body, 0)
    pltpu.sync_copy(hb, g_hbm.at[pl.ds(base, NSLICE)])
    pltpu.sync_copy(hb, acc_sh.at[pl.ds(base, NSLICE)])
    plsc.subcore_barrier()
    pltpu.sync_copy(tm_hbm.at[0, pl.ds(wid * CPT_HALF, CPT_HALF)], sidx)
    pltpu.sync_copy(tm_hbm.at[1, pl.ds(wid * CPT_HALF, CPT_HALF)], didx)

    ngroups = CPT_HALF // GROUP
    ssems = [ssem0, ssem1]

    def fire_gathers(g):
        ds_ = []
        for i in range(GROUP):
            ds_.append(pltpu.async_copy(
                g_hbm.at[sidx.at[g * GROUP + i]], rows.at[g % 2, i], gsem))
        return ds_

    def fire_scatters(g):
        ds_ = []
        for i in range(GROUP):
            ds_.append(pltpu.async_copy(
                rows.at[g % 2, i], acc_sh.at[didx.at[g * GROUP + i]],
                ssems[g % 2], add=True))
        return ds_

    gd = fire_gathers(0)
    sd = [None, None]
    for g in range(ngroups):
        for d_ in gd:
            d_.wait()
        sd[g % 2] = fire_scatters(g)
        if g + 1 < ngroups:
            if sd[(g + 1) % 2] is not None:
                for d_ in sd[(g + 1) % 2]:
                    d_.wait()
            gd = fire_gathers(g + 1)
    for s in sd:
        if s is not None:
            for d_ in s:
                d_.wait()
    plsc.subcore_barrier()
    pltpu.sync_copy(acc_sh.at[pl.ds(base, NSLICE)],
                    acc_hbm.at[cid, pl.ds(base, NSLICE)])


@functools.partial(
    pl.kernel,
    out_type=jax.ShapeDtypeStruct((NPAD, F), jnp.float32),
    mesh=_MESH,
    compiler_params=_SC_PARAMS,
    scratch_types=[
        pltpu.VMEM((NSLICE32, F), jnp.float32),
        pltpu.VMEM((NSLICE32, F), jnp.float32),
        pltpu.VMEM((NSLICE32, F), jnp.float32),
        pltpu.VMEM((NSLICE32,), jnp.float32),
        pltpu.VMEM((F,), jnp.float32),
    ],
)
def _fin_kernel(acc_hbm, g_hbm, dinv_hbm, b_hbm, out_hbm,
                a0, a1, gb, dinvb, bb):
    cid = lax.axis_index("c")
    sid = lax.axis_index("s")
    wid = cid * NS + sid
    base = wid * NSLICE32
    pltpu.sync_copy(acc_hbm.at[0, pl.ds(base, NSLICE32)], a0)
    pltpu.sync_copy(acc_hbm.at[1, pl.ds(base, NSLICE32)], a1)
    pltpu.sync_copy(g_hbm.at[pl.ds(base, NSLICE32)], gb)
    pltpu.sync_copy(dinv_hbm.at[pl.ds(base, NSLICE32)], dinvb)
    pltpu.sync_copy(b_hbm, bb)
    bv = bb[...]

    def row_body(k, carry):
        dv = dinvb[pl.ds(k * F, F)]
        for l in range(F):
            r = k * F + l
            gb[r, :] = (a0[r, :] + a1[r, :] - gb[r, :]) * dv[l] + bv
        return carry

    lax.fori_loop(0, NSLICE32 // F, row_body, 0)
    pltpu.sync_copy(gb, out_hbm.at[pl.ds(base, NSLICE32)])


@functools.partial(
    pl.kernel,
    out_type=jax.ShapeDtypeStruct((NF, F, B), jnp.float32),
    mesh=_MESH,
    compiler_params=_SC_PARAMS,
    scratch_types=[
        pltpu.VMEM((B,), jnp.int32),
        pltpu.VMEM((B, F), jnp.float32),
        pltpu.VMEM((F, B // 2), jnp.float32),
        pltpu.SemaphoreType.DMA,
        pltpu.SemaphoreType.DMA,
    ],
)
def _gather_kernel(out_hbm, xt_hbm, y_hbm, xidx, rows, slab, gsem0, gsem1):
    cid = lax.axis_index("c")
    sid = lax.axis_index("s")
    wid = cid * NS + sid
    lane = lax.iota(jnp.int32, F)
    half = B // 2

    @pl.when(wid < NF)
    def _():
        pltpu.sync_copy(xt_hbm.at[wid], xidx)
        for k in range(16):
            pltpu.async_copy(
                out_hbm.at[xidx.at[pl.ds(k * XCHUNK, XCHUNK)]],
                rows.at[pl.ds(k * XCHUNK, XCHUNK)], gsem0)
        descs1 = []
        for k in range(16, 32):
            descs1.append(pltpu.async_copy(
                out_hbm.at[xidx.at[pl.ds(k * XCHUNK, XCHUNK)]],
                rows.at[pl.ds(k * XCHUNK, XCHUNK)], gsem1))
        pltpu.make_async_copy(
            out_hbm.at[pl.ds(0, half)], rows.at[pl.ds(0, half)], gsem0).wait()

        diags = [(lane + d) & (F - 1) for d in range(F)]

        QU = 8

        def make_tbody(hh):
            def tbody(q0, carry):
                for qq in range(QU):
                    ridx = hh * half + (q0 * QU + qq) * F + lane
                    cidx = (q0 * QU + qq) * F + lane
                    for d in range(F):
                        v = plsc.load_gather(rows, [ridx, diags[d]])
                        plsc.store_scatter(slab, [diags[d], cidx], v)
                return carry
            return tbody

        lax.fori_loop(0, half // F // QU, make_tbody(0), 0)
        pltpu.sync_copy(slab, y_hbm.at[wid, :, pl.ds(0, half)])
        for d in descs1:
            d.wait()
        lax.fori_loop(0, half // F // QU, make_tbody(1), 0)
        pltpu.sync_copy(slab, y_hbm.at[wid, :, pl.ds(half, half)])


@jax.jit
def _run(features, train_mat, W, b, x):
    tmr = train_mat.reshape(2, ECHUNKS, CHUNK)
    h = _mm_call(features, W)
    deg_flat = _deg_kernel(tmr)
    g, dinv, acc_parts = _agg_kernel(deg_flat, h, tmr)
    out = _fin_kernel(acc_parts, g, dinv, b)
    y = _gather_kernel(out, x.T)
    return jnp.transpose(y, (2, 0, 1))


def kernel(features, train_mat, W, b, x):
    return _run(features, train_mat, W, b, x)


# speedup vs baseline: 1.4213x; 1.0457x over previous
import functools

import jax
import jax.numpy as jnp
from jax import lax
from jax.experimental import pallas as pl
from jax.experimental.pallas import tpu as pltpu
from jax.experimental.pallas import tpu_sc as plsc

N = 10000
D = 128
F = 16
E = 320000
B = 4096
NF = 26
NC, NS = 2, 16
NW = NC * NS
NPAD = 10240
NSLICE = NPAD // NS
NSLICE32 = NPAD // NW
CHUNK = 125
ECHUNKS = E // CHUNK
CPT_HALF = E // NW // CHUNK
CPT_FULL = E // NS // CHUNK
GROUP = 16
XCHUNK = 128
XCH_PER_TILE = B * NF // NW // XCHUNK
XROWS = B * NF // NW

_MESH = plsc.VectorSubcoreMesh(
    core_axis_name="c", subcore_axis_name="s", num_cores=NC, num_subcores=NS)
_SC_PARAMS = pltpu.CompilerParams(
    use_tc_tiling_on_sc=False, needs_layout_passes=False)


def _rsqrt16(x):
    i = plsc.bitcast(x, jnp.int32)
    y = plsc.bitcast(jnp.int32(0x5F3759DF) - (i >> 1), jnp.float32)
    for _ in range(3):
        y = y * (1.5 - 0.5 * x * y * y)
    return y


_MMBLK = 1024


def _mm_body(feat_ref, w_ref, h_ref):
    h_ref[...] = jnp.dot(feat_ref[...], w_ref[...],
                         preferred_element_type=jnp.float32)


_mm_call = pl.pallas_call(
    _mm_body,
    grid=(NPAD // _MMBLK,),
    in_specs=[
        pl.BlockSpec((_MMBLK, D), lambda i: (i, 0)),
        pl.BlockSpec((D, F), lambda i: (0, 0)),
    ],
    out_specs=pl.BlockSpec((_MMBLK, F), lambda i: (i, 0)),
    out_shape=jax.ShapeDtypeStruct((NPAD, F), jnp.float32),
)


@functools.partial(
    pl.kernel,
    out_type=jax.ShapeDtypeStruct((NC * NPAD,), jnp.float32),
    mesh=_MESH,
    compiler_params=_SC_PARAMS,
    scratch_types=[
        pltpu.VMEM((CPT_FULL, CHUNK), jnp.int32),
        pltpu.VMEM((128,), jnp.float32),
        pltpu.VMEM((NSLICE,), jnp.float32),
        pltpu.VMEM_SHARED((NPAD,), jnp.float32),
        pltpu.SemaphoreType.DMA,
    ],
)
def _deg_kernel(tm_hbm, deg_hbm, didx, ones, zbuf, deg_sh, sem):
    cid = lax.axis_index("c")
    sid = lax.axis_index("s")
    for i in range(128 // F):
        ones[pl.ds(i * F, F)] = jnp.ones((F,), jnp.float32)
    for i in range(NSLICE // F):
        zbuf[pl.ds(i * F, F)] = jnp.zeros((F,), jnp.float32)
    pltpu.sync_copy(zbuf, deg_sh.at[pl.ds(sid * NSLICE, NSLICE)])
    plsc.subcore_barrier()
    pltpu.sync_copy(tm_hbm.at[1, pl.ds(sid * CPT_FULL, CPT_FULL)], didx)

    def group_body(gi, carry):
        j0 = gi * GROUP
        descs = []
        for i in range(GROUP):
            descs.append(pltpu.async_copy(
                ones.at[pl.ds(0, CHUNK)], deg_sh.at[didx.at[j0 + i]], sem,
                add=True))
        for d in descs:
            d.wait()
        return carry

    lax.fori_loop(0, CPT_FULL // GROUP, group_body, 0)
    plsc.subcore_barrier()
    pltpu.sync_copy(deg_sh.at[pl.ds(sid * NSLICE, NSLICE)],
                    deg_hbm.at[pl.ds(cid * NPAD + sid * NSLICE, NSLICE)])


@functools.partial(
    pl.kernel,
    out_type=(
        jax.ShapeDtypeStruct((NPAD, F), jnp.float32),
        jax.ShapeDtypeStruct((NPAD,), jnp.float32),
        jax.ShapeDtypeStruct((NC, NPAD, F), jnp.float32),
    ),
    mesh=_MESH,
    compiler_params=_SC_PARAMS,
    scratch_types=[
        pltpu.VMEM((NSLICE,), jnp.float32),
        pltpu.VMEM((NSLICE,), jnp.float32),
        pltpu.VMEM((NSLICE, F), jnp.float32),
        pltpu.VMEM((CPT_HALF, CHUNK), jnp.int32),
        pltpu.VMEM((CPT_HALF, CHUNK), jnp.int32),
        pltpu.VMEM((2, GROUP, CHUNK, F), jnp.float32),
        pltpu.VMEM_SHARED((NPAD, F), jnp.float32),
        pltpu.SemaphoreType.DMA,
        pltpu.SemaphoreType.DMA,
        pltpu.SemaphoreType.DMA,
    ],
)
def _agg_kernel(deg_hbm, h_hbm, tm_hbm, g_hbm, dinv_hbm, acc_hbm,
                degb, dinvb, hb, sidx, didx, rows, acc_sh, gsem, ssem0, ssem1):
    cid = lax.axis_index("c")
    sid = lax.axis_index("s")
    wid = cid * NS + sid
    base = sid * NSLICE
    pltpu.sync_copy(deg_hbm.at[pl.ds(cid * NPAD + base, NSLICE)], degb)

    def rsqrt_body(k, carry):
        v = degb[pl.ds(k * F, F)] + 1.0
        dinvb[pl.ds(k * F, F)] = _rsqrt16(v)
        return carry

    lax.fori_loop(0, NSLICE // F, rsqrt_body, 0)
    pltpu.sync_copy(dinvb, dinv_hbm.at[pl.ds(base, NSLICE)])
    pltpu.sync_copy(h_hbm.at[pl.ds(base, NSLICE)], hb)

    def scale_body(k, carry):
        dv = dinvb[pl.ds(k * F, F)]
        for l in range(F):
            r = k * F + l
            hb[r, :] = hb[r, :] * dv[l]
        return carry

    lax.fori_loop(0, NSLICE // F, scale_body, 0)
    pltpu.sync_copy(hb, g_hbm.at[pl.ds(base, NSLICE)])
    pltpu.sync_copy(hb, acc_sh.at[pl.ds(base, NSLICE)])
    plsc.subcore_barrier()
    pltpu.sync_copy(tm_hbm.at[0, pl.ds(wid * CPT_HALF, CPT_HALF)], sidx)
    pltpu.sync_copy(tm_hbm.at[1, pl.ds(wid * CPT_HALF, CPT_HALF)], didx)

    ngroups = CPT_HALF // GROUP
    ssems = [ssem0, ssem1]

    def fire_gathers(g):
        ds_ = []
        for i in range(GROUP):
            ds_.append(pltpu.async_copy(
                g_hbm.at[sidx.at[g * GROUP + i]], rows.at[g % 2, i], gsem))
        return ds_

    def fire_scatters(g):
        ds_ = []
        for i in range(GROUP):
            ds_.append(pltpu.async_copy(
                rows.at[g % 2, i], acc_sh.at[didx.at[g * GROUP + i]],
                ssems[g % 2], add=True))
        return ds_

    gd = fire_gathers(0)
    sd = [None, None]
    for g in range(ngroups):
        for d_ in gd:
            d_.wait()
        sd[g % 2] = fire_scatters(g)
        if g + 1 < ngroups:
            if sd[(g + 1) % 2] is not None:
                for d_ in sd[(g + 1) % 2]:
                    d_.wait()
            gd = fire_gathers(g + 1)
    for s in sd:
        if s is not None:
            for d_ in s:
                d_.wait()
    plsc.subcore_barrier()
    pltpu.sync_copy(acc_sh.at[pl.ds(base, NSLICE)],
                    acc_hbm.at[cid, pl.ds(base, NSLICE)])


@functools.partial(
    pl.kernel,
    out_type=jax.ShapeDtypeStruct((NPAD, F), jnp.float32),
    mesh=_MESH,
    compiler_params=_SC_PARAMS,
    scratch_types=[
        pltpu.VMEM((NSLICE32, F), jnp.float32),
        pltpu.VMEM((NSLICE32, F), jnp.float32),
        pltpu.VMEM((NSLICE32, F), jnp.float32),
        pltpu.VMEM((NSLICE32,), jnp.float32),
        pltpu.VMEM((F,), jnp.float32),
    ],
)
def _fin_kernel(acc_hbm, g_hbm, dinv_hbm, b_hbm, out_hbm,
                a0, a1, gb, dinvb, bb):
    cid = lax.axis_index("c")
    sid = lax.axis_index("s")
    wid = cid * NS + sid
    base = wid * NSLICE32
    pltpu.sync_copy(acc_hbm.at[0, pl.ds(base, NSLICE32)], a0)
    pltpu.sync_copy(acc_hbm.at[1, pl.ds(base, NSLICE32)], a1)
    pltpu.sync_copy(g_hbm.at[pl.ds(base, NSLICE32)], gb)
    pltpu.sync_copy(dinv_hbm.at[pl.ds(base, NSLICE32)], dinvb)
    pltpu.sync_copy(b_hbm, bb)
    bv = bb[...]

    def row_body(k, carry):
        dv = dinvb[pl.ds(k * F, F)]
        for l in range(F):
            r = k * F + l
            gb[r, :] = (a0[r, :] + a1[r, :] - gb[r, :]) * dv[l] + bv
        return carry

    lax.fori_loop(0, NSLICE32 // F, row_body, 0)
    pltpu.sync_copy(gb, out_hbm.at[pl.ds(base, NSLICE32)])


@functools.partial(
    pl.kernel,
    out_type=jax.ShapeDtypeStruct((NF, F // 8, B // XCHUNK, 8, XCHUNK),
                                  jnp.float32),
    mesh=_MESH,
    compiler_params=_SC_PARAMS,
    scratch_types=[
        pltpu.VMEM((B,), jnp.int32),
        pltpu.VMEM((B, F), jnp.float32),
        pltpu.VMEM((F, B // 2), jnp.float32),
        pltpu.SemaphoreType.DMA,
        pltpu.SemaphoreType.DMA,
        pltpu.SemaphoreType.DMA,
    ],
)
def _gather_kernel(out_hbm, xt_hbm, y_hbm, xidx, rows, slab, gsem0, gsem1,
                   wsem):
    cid = lax.axis_index("c")
    sid = lax.axis_index("s")
    wid = cid * NS + sid
    lane = lax.iota(jnp.int32, F)
    half = B // 2

    @pl.when(wid < NF)
    def _():
        pltpu.sync_copy(xt_hbm.at[wid], xidx)
        for k in range(16):
            pltpu.async_copy(
                out_hbm.at[xidx.at[pl.ds(k * XCHUNK, XCHUNK)]],
                rows.at[pl.ds(k * XCHUNK, XCHUNK)], gsem0)
        descs1 = []
        for k in range(16, 32):
            descs1.append(pltpu.async_copy(
                out_hbm.at[xidx.at[pl.ds(k * XCHUNK, XCHUNK)]],
                rows.at[pl.ds(k * XCHUNK, XCHUNK)], gsem1))
        pltpu.make_async_copy(
            out_hbm.at[pl.ds(0, half)], rows.at[pl.ds(0, half)], gsem0).wait()

        diags = [(lane + d) & (F - 1) for d in range(F)]

        QU = 8

        def make_tbody(hh):
            def tbody(q0, carry):
                for qq in range(QU):
                    ridx = hh * half + (q0 * QU + qq) * F + lane
                    cidx = (q0 * QU + qq) * F + lane
                    for d in range(F):
                        v = plsc.load_gather(rows, [ridx, diags[d]])
                        plsc.store_scatter(slab, [diags[d], cidx], v)
                return carry
            return tbody

        def write_half(hh):
            ds_ = []
            for et in range(F // 8):
                for lit in range(16):
                    git = hh * 16 + lit
                    ds_.append(pltpu.async_copy(
                        slab.at[pl.ds(et * 8, 8), pl.ds(lit * XCHUNK, XCHUNK)],
                        y_hbm.at[wid, et, git], wsem))
            return ds_

        lax.fori_loop(0, half // F // QU, make_tbody(0), 0)
        wd = write_half(0)
        for d in descs1:
            d.wait()
        for d in wd:
            d.wait()
        lax.fori_loop(0, half // F // QU, make_tbody(1), 0)
        for d in write_half(1):
            d.wait()


@jax.jit
def _run(features, train_mat, W, b, x):
    tmr = train_mat.reshape(2, ECHUNKS, CHUNK)
    h = _mm_call(features, W)
    deg_flat = _deg_kernel(tmr)
    g, dinv, acc_parts = _agg_kernel(deg_flat, h, tmr)
    out = _fin_kernel(acc_parts, g, dinv, b)
    y5 = _gather_kernel(out, x.T)
    return jnp.transpose(y5, (2, 4, 0, 1, 3)).reshape(B, NF, F)


def kernel(features, train_mat, W, b, x):
    return _run(features, train_mat, W, b, x)
